# Initial kernel scaffold; baseline (speedup 1.0000x reference)
import functools

import jax
import jax.numpy as jnp
from jax import lax
from jax.experimental import pallas as pl
from jax.experimental.pallas import tpu as pltpu
from jax.experimental.pallas import tpu_sc as plsc

_VOCAB = 1000000
_MAXLEN = 200
_DIM = 32
_BATCH = 4096
_TOTAL = _BATCH * _MAXLEN
_NW = 32
_PER_TILE = _TOTAL // _NW
_GATHER = 40
_G_PER_SLOT = 20
_SLOT = _GATHER * _G_PER_SLOT
_REPS = _SLOT // _MAXLEN
_NSLOT = _PER_TILE // _SLOT
_NPAIR = _NSLOT // 2

_mesh = plsc.VectorSubcoreMesh(core_axis_name="c", subcore_axis_name="s")


@functools.partial(
    pl.kernel,
    out_type=jax.ShapeDtypeStruct((_TOTAL, _DIM), jnp.float32),
    mesh=_mesh,
    scratch_types=[
        pltpu.VMEM((_PER_TILE,), jnp.int32),
        pltpu.VMEM((_MAXLEN, _DIM), jnp.float32),
        pltpu.VMEM((_SLOT, _DIM), jnp.float32),
        pltpu.VMEM((_SLOT, _DIM), jnp.float32),
        pltpu.SemaphoreType.DMA,
        pltpu.SemaphoreType.DMA,
    ],
)
def _embed_kernel(x_hbm, tok_hbm, pos_hbm, out_hbm,
                  idx_v, pos_v, rows0, rows1, sem0, sem1):
    wid = lax.axis_index("s") * 2 + lax.axis_index("c")
    base = wid * _PER_TILE

    pltpu.sync_copy(x_hbm.at[pl.ds(base, _PER_TILE)], idx_v)
    pltpu.sync_copy(pos_hbm, pos_v)

    def fire(s, rows, sem):
        for j in range(_G_PER_SLOT):
            pltpu.async_copy(
                tok_hbm.at[idx_v.at[pl.ds(s * _SLOT + j * _GATHER, _GATHER)]],
                rows.at[pl.ds(j * _GATHER, _GATHER)],
                sem,
            )

    def drain(rows, sem):
        pltpu.make_async_copy(tok_hbm.at[pl.ds(0, _SLOT)], rows, sem).wait()

    def add_pos(rows):
        def body(r, carry):
            p0 = pos_v[r, pl.ds(0, 16)]
            p1 = pos_v[r, pl.ds(16, 16)]
            for rep in range(_REPS):
                rr = rep * _MAXLEN + r
                plsc.addupdate(rows.at[rr, pl.ds(0, 16)], p0)
                plsc.addupdate(rows.at[rr, pl.ds(16, 16)], p1)
            return carry
        lax.fori_loop(0, _MAXLEN, body, 0)

    def write(s, rows):
        pltpu.sync_copy(rows, out_hbm.at[pl.ds(base + s * _SLOT, _SLOT)])

    fire(0, rows0, sem0)

    def pair(i, carry):
        s0 = 2 * i
        s1 = s0 + 1
        fire(s1, rows1, sem1)
        drain(rows0, sem0)
        add_pos(rows0)
        write(s0, rows0)

        @pl.when(i + 1 < _NPAIR)
        def _():
            fire(s0 + 2, rows0, sem0)

        drain(rows1, sem1)
        add_pos(rows1)
        write(s1, rows1)
        return carry

    lax.fori_loop(0, _NPAIR, pair, 0)


def kernel(x, token_table, pos_table):
    x_flat = x.reshape(-1).astype(jnp.int32)
    out = _embed_kernel(x_flat, token_table, pos_table)
    return out.reshape(_BATCH, _MAXLEN, _DIM)


# speedup vs baseline: 1.4923x; 1.4923x over previous
import functools

import jax
import jax.numpy as jnp
from jax import lax
from jax.experimental import pallas as pl
from jax.experimental.pallas import tpu as pltpu
from jax.experimental.pallas import tpu_sc as plsc

_VOCAB = 1000000
_MAXLEN = 200
_DIM = 32
_BATCH = 4096
_TOTAL = _BATCH * _MAXLEN
_NW = 32
_PER_TILE = _TOTAL // _NW
_GATHER = 40
_G_PER_SLOT = 20
_SLOT = _GATHER * _G_PER_SLOT
_REPS = _SLOT // _MAXLEN
_NSLOT = _PER_TILE // _SLOT
_NPAIR = _NSLOT // 2

_mesh = plsc.VectorSubcoreMesh(core_axis_name="c", subcore_axis_name="s")


@functools.partial(
    pl.kernel,
    out_type=jax.ShapeDtypeStruct((_TOTAL, _DIM), jnp.float32),
    mesh=_mesh,
    compiler_params=pltpu.CompilerParams(use_tc_tiling_on_sc=False),
    scratch_types=[
        pltpu.VMEM((_PER_TILE,), jnp.int32),
        pltpu.VMEM((_MAXLEN, _DIM), jnp.float32),
        pltpu.VMEM((_SLOT, _DIM), jnp.float32),
        pltpu.VMEM((_SLOT, _DIM), jnp.float32),
        pltpu.SemaphoreType.DMA,
        pltpu.SemaphoreType.DMA,
    ],
)
def _embed_kernel(x_hbm, tok_hbm, pos_hbm, out_hbm,
                  idx_v, pos_v, rows0, rows1, sem0, sem1):
    wid = lax.axis_index("s") * 2 + lax.axis_index("c")
    base = wid * _PER_TILE

    pltpu.sync_copy(x_hbm.at[pl.ds(base, _PER_TILE)], idx_v)
    pltpu.sync_copy(pos_hbm, pos_v)

    def fire(s, rows, sem):
        for j in range(_G_PER_SLOT):
            pltpu.async_copy(
                tok_hbm.at[idx_v.at[pl.ds(s * _SLOT + j * _GATHER, _GATHER)]],
                rows.at[pl.ds(j * _GATHER, _GATHER)],
                sem,
            )

    def drain(rows, sem):
        pltpu.make_async_copy(tok_hbm.at[pl.ds(0, _SLOT)], rows, sem).wait()

    def add_pos(rows):
        def body(r, carry):
            p0 = pos_v[r, pl.ds(0, 16)]
            p1 = pos_v[r, pl.ds(16, 16)]
            for rep in range(_REPS):
                rr = rep * _MAXLEN + r
                plsc.addupdate(rows.at[rr, pl.ds(0, 16)], p0)
                plsc.addupdate(rows.at[rr, pl.ds(16, 16)], p1)
            return carry
        lax.fori_loop(0, _MAXLEN, body, 0)

    def write(s, rows):
        pltpu.sync_copy(rows, out_hbm.at[pl.ds(base + s * _SLOT, _SLOT)])

    fire(0, rows0, sem0)

    def pair(i, carry):
        s0 = 2 * i
        s1 = s0 + 1
        fire(s1, rows1, sem1)
        drain(rows0, sem0)
        add_pos(rows0)
        write(s0, rows0)

        @pl.when(i + 1 < _NPAIR)
        def _():
            fire(s0 + 2, rows0, sem0)

        drain(rows1, sem1)
        add_pos(rows1)
        write(s1, rows1)
        return carry

    lax.fori_loop(0, _NPAIR, pair, 0)


def kernel(x, token_table, pos_table):
    x_flat = x.reshape(-1).astype(jnp.int32)
    out = _embed_kernel(x_flat, token_table, pos_table)
    return out.reshape(_BATCH, _MAXLEN, _DIM)
